# Initial kernel scaffold; baseline (speedup 1.0000x reference)
import jax
import jax.numpy as jnp
from jax.experimental import pallas as pl


def kernel(logits, m_list, target):
    raise NotImplementedError("write your pallas kernel here")


# speedup vs baseline: 4.6933x; 4.6933x over previous
import functools

import jax
import jax.numpy as jnp
from jax import lax
from jax.experimental import pallas as pl

_S = 30.0


def _ldam_body(logits_ref, target_ref, m_ref, out_ref, *, rows, n_classes, batch):
    i = pl.program_id(0)

    x = logits_ref[...]
    t = target_ref[0, 0, :]
    m = m_ref[0, :]

    lane = lax.broadcasted_iota(jnp.int32, (rows, n_classes), 1)
    is_t = lane == t[:, None]

    l_t = jnp.sum(jnp.where(is_t, x, 0.0), axis=1, keepdims=True)
    m_row = jnp.sum(jnp.where(is_t, m[None, :], 0.0), axis=1, keepdims=True)

    row_max = jnp.max(x, axis=1, keepdims=True)
    e = jnp.exp(x - row_max)
    se_others = jnp.sum(jnp.where(is_t, 0.0, e), axis=1, keepdims=True)

    a = l_t - _S * m_row
    se_adj = se_others + jnp.exp(a - row_max)
    nll = row_max + jnp.log(se_adj) - a

    partial = jnp.sum(nll, axis=(0, 1), keepdims=True) * (1.0 / batch)

    @pl.when(i == 0)
    def _():
        out_ref[...] = jnp.zeros_like(out_ref)

    out_ref[...] += partial


def kernel(logits, m_list, target):
    batch, n_classes = logits.shape
    rows = 512
    grid = batch // rows

    target3 = target.reshape(grid, 1, rows)
    m2 = m_list.reshape(1, n_classes)

    body = functools.partial(_ldam_body, rows=rows, n_classes=n_classes, batch=batch)
    out = pl.pallas_call(
        body,
        grid=(grid,),
        in_specs=[
            pl.BlockSpec((rows, n_classes), lambda i: (i, 0)),
            pl.BlockSpec((1, 1, rows), lambda i: (i, 0, 0)),
            pl.BlockSpec((1, n_classes), lambda i: (0, 0)),
        ],
        out_specs=pl.BlockSpec((1, 1), lambda i: (0, 0)),
        out_shape=jax.ShapeDtypeStruct((1, 1), jnp.float32),
    )(logits, target3, m2)
    return out[0, 0]


# speedup vs baseline: 6.1701x; 1.3147x over previous
import functools

import jax
import jax.numpy as jnp
from jax import lax
from jax.experimental import pallas as pl

_S = 30.0


def _ldam_body(logits_ref, target_ref, m_ref, out_ref, *, rows, n_classes, batch):
    i = pl.program_id(0)

    x = logits_ref[...]
    t = target_ref[0, 0, :]
    m = m_ref[0, :]

    lane = lax.broadcasted_iota(jnp.int32, (rows, n_classes), 1)
    is_t = lane == t[:, None]

    l_t = jnp.sum(jnp.where(is_t, x, 0.0), axis=1, keepdims=True)
    m_row = jnp.sum(jnp.where(is_t, m[None, :], 0.0), axis=1, keepdims=True)

    row_max = jnp.max(x, axis=1, keepdims=True)
    e = jnp.exp(x - row_max)
    se_others = jnp.sum(jnp.where(is_t, 0.0, e), axis=1, keepdims=True)

    a = l_t - _S * m_row
    se_adj = se_others + jnp.exp(a - row_max)
    nll = row_max + jnp.log(se_adj) - a

    partial = jnp.sum(nll, axis=(0, 1), keepdims=True) * (1.0 / batch)

    @pl.when(i == 0)
    def _():
        out_ref[...] = jnp.zeros_like(out_ref)

    out_ref[...] += partial


def kernel(logits, m_list, target):
    batch, n_classes = logits.shape
    rows = 1024
    grid = batch // rows

    target3 = target.reshape(grid, 1, rows)
    m2 = m_list.reshape(1, n_classes)

    body = functools.partial(_ldam_body, rows=rows, n_classes=n_classes, batch=batch)
    out = pl.pallas_call(
        body,
        grid=(grid,),
        in_specs=[
            pl.BlockSpec((rows, n_classes), lambda i: (i, 0)),
            pl.BlockSpec((1, 1, rows), lambda i: (i, 0, 0)),
            pl.BlockSpec((1, n_classes), lambda i: (0, 0)),
        ],
        out_specs=pl.BlockSpec((1, 1), lambda i: (0, 0)),
        out_shape=jax.ShapeDtypeStruct((1, 1), jnp.float32),
    )(logits, target3, m2)
    return out[0, 0]


# speedup vs baseline: 6.8020x; 1.1024x over previous
import functools

import jax
import jax.numpy as jnp
from jax import lax
from jax.experimental import pallas as pl

_S = 30.0


def _ldam_body(logits_ref, target_ref, m_ref, out_ref, *, rows, n_classes, batch):
    i = pl.program_id(0)

    x = logits_ref[...]
    t = target_ref[0, 0, :]
    m = m_ref[0, :]

    lane = lax.broadcasted_iota(jnp.int32, (rows, n_classes), 1)
    is_t = lane == t[:, None]

    l_t = jnp.sum(jnp.where(is_t, x, 0.0), axis=1, keepdims=True)
    m_row = jnp.sum(jnp.where(is_t, m[None, :], 0.0), axis=1, keepdims=True)

    row_max = jnp.max(x, axis=1, keepdims=True)
    e = jnp.exp(x - row_max)
    se_others = jnp.sum(jnp.where(is_t, 0.0, e), axis=1, keepdims=True)

    a = l_t - _S * m_row
    se_adj = se_others + jnp.exp(a - row_max)
    nll = row_max + jnp.log(se_adj) - a

    partial = jnp.sum(nll, axis=(0, 1), keepdims=True) * (1.0 / batch)

    @pl.when(i == 0)
    def _():
        out_ref[...] = jnp.zeros_like(out_ref)

    out_ref[...] += partial


def kernel(logits, m_list, target):
    batch, n_classes = logits.shape
    rows = 2048
    grid = batch // rows

    target3 = target.reshape(grid, 1, rows)
    m2 = m_list.reshape(1, n_classes)

    body = functools.partial(_ldam_body, rows=rows, n_classes=n_classes, batch=batch)
    out = pl.pallas_call(
        body,
        grid=(grid,),
        in_specs=[
            pl.BlockSpec((rows, n_classes), lambda i: (i, 0)),
            pl.BlockSpec((1, 1, rows), lambda i: (i, 0, 0)),
            pl.BlockSpec((1, n_classes), lambda i: (0, 0)),
        ],
        out_specs=pl.BlockSpec((1, 1), lambda i: (0, 0)),
        out_shape=jax.ShapeDtypeStruct((1, 1), jnp.float32),
    )(logits, target3, m2)
    return out[0, 0]


# speedup vs baseline: 6.9238x; 1.0179x over previous
import functools

import jax
import jax.numpy as jnp
from jax import lax
from jax.experimental import pallas as pl

_S = 30.0


def _ldam_body(logits_ref, target_ref, m_ref, out_ref, *, rows, n_classes, batch):
    i = pl.program_id(0)

    x = logits_ref[...]
    t = target_ref[0, 0, :]
    m = m_ref[0, :]

    lane = lax.broadcasted_iota(jnp.int32, (rows, n_classes), 1)
    is_t = lane == t[:, None]

    l_t = jnp.sum(jnp.where(is_t, x, 0.0), axis=1, keepdims=True)
    m_row = jnp.sum(jnp.where(is_t, m[None, :], 0.0), axis=1, keepdims=True)

    row_max = jnp.max(x, axis=1, keepdims=True)
    e = jnp.exp(x - row_max)
    se_others = jnp.sum(jnp.where(is_t, 0.0, e), axis=1, keepdims=True)

    a = l_t - _S * m_row
    se_adj = se_others + jnp.exp(a - row_max)
    nll = row_max + jnp.log(se_adj) - a

    partial = jnp.sum(nll, axis=(0, 1), keepdims=True) * (1.0 / batch)

    @pl.when(i == 0)
    def _():
        out_ref[...] = jnp.zeros_like(out_ref)

    out_ref[...] += partial


def kernel(logits, m_list, target):
    batch, n_classes = logits.shape
    rows = 4096
    grid = batch // rows

    target3 = target.reshape(grid, 1, rows)
    m2 = m_list.reshape(1, n_classes)

    body = functools.partial(_ldam_body, rows=rows, n_classes=n_classes, batch=batch)
    out = pl.pallas_call(
        body,
        grid=(grid,),
        in_specs=[
            pl.BlockSpec((rows, n_classes), lambda i: (i, 0)),
            pl.BlockSpec((1, 1, rows), lambda i: (i, 0, 0)),
            pl.BlockSpec((1, n_classes), lambda i: (0, 0)),
        ],
        out_specs=pl.BlockSpec((1, 1), lambda i: (0, 0)),
        out_shape=jax.ShapeDtypeStruct((1, 1), jnp.float32),
    )(logits, target3, m2)
    return out[0, 0]


# speedup vs baseline: 7.9821x; 1.1529x over previous
import functools

import jax
import jax.numpy as jnp
from jax import lax
from jax.experimental import pallas as pl

_S = 30.0


def _ldam_body(logits_ref, target_ref, m_ref, out_ref, *, rows, n_classes, batch):
    i = pl.program_id(0)

    x = logits_ref[...]
    t = target_ref[0, 0, :]
    m = m_ref[0, :]

    lane = lax.broadcasted_iota(jnp.int32, (rows, n_classes), 1)
    tmask = (lane == t[:, None]).astype(jnp.float32)

    ones_col = jnp.ones((n_classes, 1), jnp.float32)
    m_col = m.reshape(n_classes, 1)

    row_max = jnp.max(x, axis=1, keepdims=True)
    e = jnp.exp(x - row_max)

    l_t = jnp.dot(x * tmask, ones_col, preferred_element_type=jnp.float32)
    m_row = jnp.dot(tmask, m_col, preferred_element_type=jnp.float32)
    se_all = jnp.dot(e, ones_col, preferred_element_type=jnp.float32)

    a = l_t - _S * m_row
    e_t = jnp.exp(l_t - row_max)
    se_adj = se_all - e_t + jnp.exp(a - row_max)
    nll = row_max + jnp.log(se_adj) - a

    partial = jnp.sum(nll, axis=(0, 1), keepdims=True) * (1.0 / batch)

    @pl.when(i == 0)
    def _():
        out_ref[...] = jnp.zeros_like(out_ref)

    out_ref[...] += partial


def kernel(logits, m_list, target):
    batch, n_classes = logits.shape
    rows = 4096
    grid = batch // rows

    target3 = target.reshape(grid, 1, rows)
    m2 = m_list.reshape(1, n_classes)

    body = functools.partial(_ldam_body, rows=rows, n_classes=n_classes, batch=batch)
    out = pl.pallas_call(
        body,
        grid=(grid,),
        in_specs=[
            pl.BlockSpec((rows, n_classes), lambda i: (i, 0)),
            pl.BlockSpec((1, 1, rows), lambda i: (i, 0, 0)),
            pl.BlockSpec((1, n_classes), lambda i: (0, 0)),
        ],
        out_specs=pl.BlockSpec((1, 1), lambda i: (0, 0)),
        out_shape=jax.ShapeDtypeStruct((1, 1), jnp.float32),
    )(logits, target3, m2)
    return out[0, 0]
